# baseline (speedup 1.0000x reference)
import functools

import jax
import jax.numpy as jnp
from jax.experimental import pallas as pl
from jax.experimental.pallas import tpu as pltpu

_T = 8


def _maps_kernel(wt_ref, w2_ref, o_ref, *, h, w, np):
    f32 = jnp.float32
    yy = jax.lax.broadcasted_iota(jnp.int32, (h, w), 0)
    xx = jax.lax.broadcasted_iota(jnp.int32, (h, w), 1)
    acc = jnp.zeros((np, h, w), f32)
    tap = 0
    for d in (1, 2, 3):
        for a in range(3):
            dy = (a - 1) * d
            for b in range(3):
                dx = (b - 1) * d
                plane = ((yy >= max(0, dy)) & (yy < h + min(0, dy)) &
                         (xx >= max(0, dx)) & (xx < w + min(0, dx)))
                acc = acc + wt_ref[tap][:, :, None] * plane.astype(f32)[None]
                tap += 1
    o_ref[0:np] = acc

    def shift2(v, oy, ox):
        if oy > 0:
            v = jnp.concatenate([v[:, oy:, :], jnp.zeros_like(v[:, :oy, :])], 1)
        elif oy < 0:
            v = jnp.concatenate([jnp.zeros_like(v[:, oy:, :]), v[:, :oy, :]], 1)
        if ox > 0:
            v = jnp.concatenate([v[:, :, ox:], jnp.zeros_like(v[:, :, :ox])], 2)
        elif ox < 0:
            v = jnp.concatenate([jnp.zeros_like(v[:, :, ox:]), v[:, :, :ox]], 2)
        return v

    mc = jnp.zeros((np, h, w), f32)
    for ky in range(3):
        for kx in range(3):
            shifted = shift2(acc, ky - 1, kx - 1)
            mc = mc + w2_ref[ky * 3 + kx][:, :, None] * shifted
    o_ref[np:2 * np] = mc


def _fused_kernel(m_ref, w1_ref, b1_ref, b123_ref, w4_ref, b4_ref,
                  x_ref, o_hbm, obuf, osem, *, t, cr, inv_hw):
    m = m_ref[...]
    w1 = w1_ref[...]
    b1 = b1_ref[...]
    cols = []
    for f in range(t):
        feat = jnp.dot(w1, x_ref[f], preferred_element_type=jnp.float32) + b1
        s = jnp.sum(m * feat[None], axis=2)
        cols.append(jnp.sum(s, axis=1, keepdims=True))
    P = jnp.concatenate(cols, axis=1)
    Pl = P[0:cr]
    Pr = P[cr:2 * cr]
    Pcl0 = P[2 * cr:3 * cr, 0:1]
    Pcr7 = P[3 * cr:4 * cr, t - 1:t]
    Pr1 = jnp.concatenate([Pr[:, 1:], jnp.zeros((cr, 1), jnp.float32)], axis=1)
    mask = (jax.lax.broadcasted_iota(jnp.int32, (1, t), 1)
            < t - 1).astype(jnp.float32)
    b123c = b123_ref[...]
    pooled_l = mask * (Pcl0 - Pl) * inv_hw + b123c
    pooled_r = mask * (Pcr7 - Pr1) * inv_hw + b123c
    z_l = jnp.dot(w4_ref[...], pooled_l,
                  preferred_element_type=jnp.float32) + b4_ref[...]
    z_r = jnp.dot(w4_ref[...], pooled_r,
                  preferred_element_type=jnp.float32) + b4_ref[...]
    w_mod = 0.5 * (jax.nn.sigmoid(z_l) + jax.nn.sigmoid(z_r)) - 0.5
    i = pl.program_id(0)

    def copy_for(f):
        return pltpu.make_async_copy(
            obuf.at[f % 2], o_hbm.at[i, f], osem.at[f % 2])

    for f in range(t):
        if f >= 2:
            copy_for(f - 2).wait()
        obuf[f % 2] = x_ref[f] * (1.0 + w_mod[:, f:f + 1])
        copy_for(f).start()
    copy_for(t - 2).wait()
    copy_for(t - 1).wait()


def kernel(x, w1_eff, b1_eff, conv2_flat, w_lr_flat, b123, w4_eff, b4_eff):
    nt, c, h, w = x.shape
    cr = w1_eff.shape[0]
    hw = h * w
    n = nt // _T
    f32 = jnp.float32
    x_flat = x.reshape(nt, c, hw)
    npl = 2 * cr * cr

    wt = w_lr_flat.reshape(2, 3, cr, cr, 3, 3).transpose(1, 4, 5, 0, 2, 3)
    wt = wt.reshape(27, npl, 1)
    w2f = conv2_flat.reshape(cr, 3, 3)[:, ::-1, ::-1]
    w2v = jnp.broadcast_to(w2f.transpose(1, 2, 0).reshape(9, 1, 1, cr),
                           (9, 2, cr, cr)).reshape(9, npl, 1)

    maps = pl.pallas_call(
        functools.partial(_maps_kernel, h=h, w=w, np=npl),
        out_shape=jax.ShapeDtypeStruct((2 * npl, h, w), f32),
    )(wt, w2v).reshape(4 * cr, cr, hw)

    nk = 4 * cr
    x_seg = x_flat.reshape(n, _T, c, hw)

    out = pl.pallas_call(
        functools.partial(_fused_kernel, t=_T, cr=cr, inv_hw=1.0 / float(hw)),
        out_shape=jax.ShapeDtypeStruct((n, _T, c, hw), f32),
        grid=(n,),
        in_specs=[
            pl.BlockSpec((nk, cr, hw), lambda i: (0, 0, 0)),
            pl.BlockSpec((cr, c), lambda i: (0, 0)),
            pl.BlockSpec((cr, 1), lambda i: (0, 0)),
            pl.BlockSpec((cr, 1), lambda i: (0, 0)),
            pl.BlockSpec((c, cr), lambda i: (0, 0)),
            pl.BlockSpec((c, 1), lambda i: (0, 0)),
            pl.BlockSpec((None, _T, c, hw), lambda i: (i, 0, 0, 0)),
        ],
        out_specs=pl.BlockSpec(memory_space=pltpu.MemorySpace.HBM),
        scratch_shapes=[
            pltpu.VMEM((2, c, hw), f32),
            pltpu.SemaphoreType.DMA((2,)),
        ],
        compiler_params=pltpu.CompilerParams(dimension_semantics=("parallel",)),
    )(maps, w1_eff, b1_eff, b123.reshape(cr, 1), w4_eff, b4_eff, x_seg)

    return out.reshape(nt, c, h, w)


# speedup vs baseline: 1.0614x; 1.0614x over previous
import functools

import jax
import jax.numpy as jnp
from jax.experimental import pallas as pl
from jax.experimental.pallas import tpu as pltpu

_T = 8


def _maps_kernel(wt_ref, w2_ref, o_ref, *, h, w, np):
    f32 = jnp.float32
    yy = jax.lax.broadcasted_iota(jnp.int32, (h, w), 0)
    xx = jax.lax.broadcasted_iota(jnp.int32, (h, w), 1)
    acc = jnp.zeros((np, h, w), f32)
    tap = 0
    for d in (1, 2, 3):
        for a in range(3):
            dy = (a - 1) * d
            for b in range(3):
                dx = (b - 1) * d
                plane = ((yy >= max(0, dy)) & (yy < h + min(0, dy)) &
                         (xx >= max(0, dx)) & (xx < w + min(0, dx)))
                acc = acc + wt_ref[tap][:, :, None] * plane.astype(f32)[None]
                tap += 1
    o_ref[0:np] = acc

    def shift2(v, oy, ox):
        if oy > 0:
            v = jnp.concatenate([v[:, oy:, :], jnp.zeros_like(v[:, :oy, :])], 1)
        elif oy < 0:
            v = jnp.concatenate([jnp.zeros_like(v[:, oy:, :]), v[:, :oy, :]], 1)
        if ox > 0:
            v = jnp.concatenate([v[:, :, ox:], jnp.zeros_like(v[:, :, :ox])], 2)
        elif ox < 0:
            v = jnp.concatenate([jnp.zeros_like(v[:, :, ox:]), v[:, :, :ox]], 2)
        return v

    mc = jnp.zeros((np, h, w), f32)
    for ky in range(3):
        for kx in range(3):
            shifted = shift2(acc, ky - 1, kx - 1)
            mc = mc + w2_ref[ky * 3 + kx][:, :, None] * shifted
    o_ref[np:2 * np] = mc


def _fused_kernel(m_ref, w1_ref, b1_ref, b123_ref, w4_ref, b4_ref,
                  x_ref, o_ref, *, t, cr, inv_hw):
    m = m_ref[...]
    w1 = w1_ref[...]
    b1 = b1_ref[...]
    cols = []
    for f in range(t):
        feat = jnp.dot(w1, x_ref[f], preferred_element_type=jnp.float32) + b1
        s = jnp.sum(m * feat[None], axis=2)
        cols.append(jnp.sum(s, axis=1, keepdims=True))
    P = jnp.concatenate(cols, axis=1)
    Pl = P[0:cr]
    Pr = P[cr:2 * cr]
    Pcl0 = P[2 * cr:3 * cr, 0:1]
    Pcr7 = P[3 * cr:4 * cr, t - 1:t]
    Pr1 = jnp.concatenate([Pr[:, 1:], jnp.zeros((cr, 1), jnp.float32)], axis=1)
    mask = (jax.lax.broadcasted_iota(jnp.int32, (1, t), 1)
            < t - 1).astype(jnp.float32)
    b123c = b123_ref[...]
    pooled_l = mask * (Pcl0 - Pl) * inv_hw + b123c
    pooled_r = mask * (Pcr7 - Pr1) * inv_hw + b123c
    z_l = jnp.dot(w4_ref[...], pooled_l,
                  preferred_element_type=jnp.float32) + b4_ref[...]
    z_r = jnp.dot(w4_ref[...], pooled_r,
                  preferred_element_type=jnp.float32) + b4_ref[...]
    w_mod = 0.5 * (jax.nn.sigmoid(z_l) + jax.nn.sigmoid(z_r)) - 0.5
    for f in range(t):
        o_ref[f] = x_ref[f] * (1.0 + w_mod[:, f:f + 1])


def kernel(x, w1_eff, b1_eff, conv2_flat, w_lr_flat, b123, w4_eff, b4_eff):
    nt, c, h, w = x.shape
    cr = w1_eff.shape[0]
    hw = h * w
    n = nt // _T
    f32 = jnp.float32
    x_flat = x.reshape(nt, c, hw)
    npl = 2 * cr * cr

    wt = w_lr_flat.reshape(2, 3, cr, cr, 3, 3).transpose(1, 4, 5, 0, 2, 3)
    wt = wt.reshape(27, npl, 1)
    w2f = conv2_flat.reshape(cr, 3, 3)[:, ::-1, ::-1]
    w2v = jnp.broadcast_to(w2f.transpose(1, 2, 0).reshape(9, 1, 1, cr),
                           (9, 2, cr, cr)).reshape(9, npl, 1)

    maps = pl.pallas_call(
        functools.partial(_maps_kernel, h=h, w=w, np=npl),
        out_shape=jax.ShapeDtypeStruct((2 * npl, h, w), f32),
    )(wt, w2v).reshape(4 * cr, cr, hw)

    nk = 4 * cr
    x_seg = x_flat.reshape(n, _T, c, hw)

    out = pl.pallas_call(
        functools.partial(_fused_kernel, t=_T, cr=cr, inv_hw=1.0 / float(hw)),
        out_shape=jax.ShapeDtypeStruct((n, _T, c, hw), f32),
        grid=(n,),
        in_specs=[
            pl.BlockSpec((nk, cr, hw), lambda i: (0, 0, 0)),
            pl.BlockSpec((cr, c), lambda i: (0, 0)),
            pl.BlockSpec((cr, 1), lambda i: (0, 0)),
            pl.BlockSpec((cr, 1), lambda i: (0, 0)),
            pl.BlockSpec((c, cr), lambda i: (0, 0)),
            pl.BlockSpec((c, 1), lambda i: (0, 0)),
            pl.BlockSpec((None, _T, c, hw), lambda i: (i, 0, 0, 0)),
        ],
        out_specs=pl.BlockSpec((None, _T, c, hw), lambda i: (i, 0, 0, 0)),
        compiler_params=pltpu.CompilerParams(
            dimension_semantics=("parallel",),
            vmem_limit_bytes=56 * 1024 * 1024),
    )(maps, w1_eff, b1_eff, b123.reshape(cr, 1), w4_eff, b4_eff, x_seg)

    return out.reshape(nt, c, h, w)
